# baseline (speedup 1.0000x reference)
import functools

import jax
import jax.numpy as jnp
from jax import lax
from jax.experimental import pallas as pl
from jax.experimental.pallas import tpu as pltpu
from jax.experimental.pallas import tpu_sc as plsc

_NW = 32
_LANES = 16
_C = 32


def kernel(x, table):
    B, S, E = x.shape
    R = B * S
    rpw = R // _NW
    nch = rpw // _C
    cw = _C * E
    mesh = plsc.VectorSubcoreMesh(core_axis_name="c", subcore_axis_name="s")

    @functools.partial(
        pl.kernel,
        mesh=mesh,
        out_type=jax.ShapeDtypeStruct((R * E,), jnp.float32),
        scratch_types=[
            pltpu.VMEM((cw,), jnp.float32),
            pltpu.VMEM((cw,), jnp.float32),
            pltpu.VMEM((cw,), jnp.float32),
            pltpu.SemaphoreType.DMA,
            pltpu.SemaphoreType.DMA,
            pltpu.SemaphoreType.DMA,
            pltpu.SemaphoreType.DMA,
            pltpu.SemaphoreType.DMA,
        ],
    )
    def sc_add(x_hbm, t_hbm, o_hbm, xb0, xb1, tb, sx0, sx1, st, so0, so1):
        wid = lax.axis_index("s") * 2 + lax.axis_index("c")
        row0 = wid * rpw
        x0 = row0 * E
        t0 = lax.rem(row0, S) * E
        xb = (xb0, xb1)
        sx = (sx0, sx1)
        so = (so0, so1)

        def start_x(g, b):
            pltpu.async_copy(x_hbm.at[pl.ds(x0 + g * cw, cw)], xb[b], sx[b])

        def wait_x(g, b):
            pltpu.make_async_copy(
                x_hbm.at[pl.ds(x0 + g * cw, cw)], xb[b], sx[b]
            ).wait()

        def start_t(g):
            pltpu.async_copy(t_hbm.at[pl.ds(t0 + g * cw, cw)], tb, st)

        def wait_t(g):
            pltpu.make_async_copy(t_hbm.at[pl.ds(t0 + g * cw, cw)], tb, st).wait()

        def wait_out(g, b):
            pltpu.make_async_copy(
                xb[b], o_hbm.at[pl.ds(x0 + g * cw, cw)], so[b]
            ).wait()

        start_x(0, 0)
        start_t(0)

        def pair(gi, carry):
            for b in (0, 1):
                g = gi * 2 + b

                @pl.when(g + 1 < nch)
                def _():
                    @pl.when(g >= 1)
                    def _():
                        wait_out(g - 1, 1 - b)

                    start_x(g + 1, 1 - b)

                wait_x(g, b)
                wait_t(g)

                def add16(i, c):
                    sl = pl.ds(i * _LANES, _LANES)
                    xb[b][sl] = xb[b][sl] + tb[sl]
                    return c

                lax.fori_loop(0, cw // _LANES, add16, 0, unroll=8)

                @pl.when(g + 1 < nch)
                def _():
                    start_t(g + 1)

                pltpu.async_copy(xb[b], o_hbm.at[pl.ds(x0 + g * cw, cw)], so[b])
            return carry

        lax.fori_loop(0, nch // 2, pair, 0)
        wait_out(nch - 2, 0)
        wait_out(nch - 1, 1)

    out = sc_add(x.reshape(R * E), table.reshape(S * E))
    return out.reshape(B, S, E)


# speedup vs baseline: 7.0717x; 7.0717x over previous
import jax
import jax.numpy as jnp
from jax.experimental import pallas as pl


def _add_kernel(x_ref, t_ref, o_ref):
    o_ref[...] = x_ref[...] + t_ref[...][None, :, :]


def kernel(x, table):
    B, S, E = x.shape
    SB = 512
    grid = (S // SB,)
    return pl.pallas_call(
        _add_kernel,
        grid=grid,
        in_specs=[
            pl.BlockSpec((B, SB, E), lambda j: (0, j, 0)),
            pl.BlockSpec((SB, E), lambda j: (j, 0)),
        ],
        out_specs=pl.BlockSpec((B, SB, E), lambda j: (0, j, 0)),
        out_shape=jax.ShapeDtypeStruct((B, S, E), x.dtype),
    )(x, table[:S])
